# baseline (speedup 1.0000x reference)
import functools

import jax
import jax.numpy as jnp
from jax import lax
from jax.experimental import pallas as pl
from jax.experimental.pallas import tpu as pltpu
from jax.experimental.pallas import tpu_sc as plsc

NG = 40
NPGR = 250
PAD = 256
DIN = 128
HID = 128
KC = 16
EDGES = 320000

GHALF = NG // 2
SLOTS_HALF = GHALF * PAD * PAD
HALFW = GHALF * PAD * 128
DUMP = 8192
SC_WORDS = SLOTS_HALF + DUMP
NSUB = 16
CHUNK = 128
FEDGES = 4 * CHUNK
NFETCH = 39
EPT = NFETCH * FEDGES
ZCH = SC_WORDS // NSUB // 8
WB = SLOTS_HALF // NSUB
WCH = 8192
MAGIC = 134218


def _adj_body(e_hbm, out_hbm, acc_sh, eb0, eb1, idx0, idx1, idx2, idx3,
              ones_v, zer_v, sem_e0, sem_e1, sem_s, sem_z):
    c = lax.axis_index("c")
    s = lax.axis_index("s")
    idxs = (idx0, idx1, idx2, idx3)

    def _fill_z(i, _):
        zer_v[pl.ds(i * 16, 16)] = jnp.zeros((16,), jnp.float32)
        return ()
    lax.fori_loop(0, ZCH // 16, _fill_z, ())
    for i in range(CHUNK // 16):
        ones_v[pl.ds(i * 16, 16)] = jnp.ones((16,), jnp.float32)

    for k in range(8):
        pltpu.async_copy(zer_v, acc_sh.at[pl.ds(s * (8 * ZCH) + k * ZCH, ZCH)], sem_z)
    for k in range(8):
        pltpu.make_async_copy(zer_v, acc_sh.at[pl.ds(s * (8 * ZCH) + k * ZCH, ZCH)], sem_z).wait()
    plsc.subcore_barrier()

    glo = c * GHALF
    off_half = c * HALFW
    base = s * EPT

    def _start_fetch(f, eb, sem):
        pltpu.async_copy(e_hbm.at[0, pl.ds(base + f * FEDGES, FEDGES)],
                         eb.at[pl.ds(0, FEDGES)], sem)
        pltpu.async_copy(e_hbm.at[1, pl.ds(base + f * FEDGES, FEDGES)],
                         eb.at[pl.ds(FEDGES, FEDGES)], sem)

    def _wait_fetch(eb, sem):
        pltpu.make_async_copy(e_hbm.at[0, pl.ds(0, FEDGES)],
                              eb.at[pl.ds(0, FEDGES)], sem).wait()
        pltpu.make_async_copy(e_hbm.at[1, pl.ds(0, FEDGES)],
                              eb.at[pl.ds(FEDGES, FEDGES)], sem).wait()

    def _scatter_chunk(eb, koff, idx_c):
        for l in range(CHUNK // 16):
            sv = eb[pl.ds(koff + l * 16, 16)]
            dv = eb[pl.ds(FEDGES + koff + l * 16, 16)]
            g = lax.shift_right_logical(sv * MAGIC, 25)
            pd = dv - g * 250
            t = lax.shift_right_logical(pd, 7)
            flat = (t * HALFW + sv * 128 + g * 768 + (pd & 127)) - off_half
            ok = (g >= glo) & (g < glo + GHALF)
            idx_c[pl.ds(l * 16, 16)] = jnp.where(
                ok, flat, SLOTS_HALF + (sv & (DUMP - 1)))
        pltpu.async_copy(ones_v, acc_sh.at[idx_c], sem_s, add=True)

    def _process(eb, sem, f_next):
        _wait_fetch(eb, sem)
        for k in range(4):
            _scatter_chunk(eb, k * CHUNK, idxs[k])
        for k in range(4):
            pltpu.make_async_copy(ones_v, acc_sh.at[idxs[k]], sem_s).wait()
        _start_fetch(f_next, eb, sem)

    _start_fetch(0, eb0, sem_e0)
    _start_fetch(1, eb1, sem_e1)

    def _pair(p, _):
        f0 = 2 * p
        _process(eb0, sem_e0, jnp.minimum(f0 + 2, NFETCH - 1))
        _process(eb1, sem_e1, jnp.minimum(f0 + 3, NFETCH - 1))
        return ()
    lax.fori_loop(0, (NFETCH - 1) // 2, _pair, ())

    _process(eb0, sem_e0, NFETCH - 1)
    _wait_fetch(eb0, sem_e0)
    _wait_fetch(eb1, sem_e1)

    @pl.when(s < 4)
    def _tail():
        toff = NSUB * EPT + s * CHUNK
        pltpu.sync_copy(e_hbm.at[0, pl.ds(toff, CHUNK)], eb0.at[pl.ds(0, CHUNK)])
        pltpu.sync_copy(e_hbm.at[1, pl.ds(toff, CHUNK)],
                        eb0.at[pl.ds(FEDGES, CHUNK)])
        _scatter_chunk(eb0, 0, idx0)
        pltpu.make_async_copy(ones_v, acc_sh.at[idx0], sem_s).wait()

    plsc.subcore_barrier()

    WHH = WB // 2
    def _wb_ranges():
        for t in range(2):
            for k in range(WHH // WCH):
                loc = t * HALFW + s * WHH + k * WCH
                glob = t * (2 * HALFW) + c * HALFW + s * WHH + k * WCH
                yield loc, glob
    for loc, glob in _wb_ranges():
        pltpu.async_copy(acc_sh.at[pl.ds(loc, WCH)],
                         out_hbm.at[pl.ds(glob, WCH)], sem_z)
    for loc, glob in _wb_ranges():
        pltpu.make_async_copy(acc_sh.at[pl.ds(loc, WCH)],
                              out_hbm.at[pl.ds(glob, WCH)], sem_z).wait()


@functools.cache
def _adj_build_fn():
    return pl.kernel(
        _adj_body,
        out_type=jax.ShapeDtypeStruct((NG * PAD * PAD,), jnp.float32),
        mesh=plsc.VectorSubcoreMesh(core_axis_name="c", subcore_axis_name="s",
                                    num_cores=2, num_subcores=NSUB),
        scratch_types=[
            pltpu.VMEM_SHARED((SC_WORDS,), jnp.float32),
            pltpu.VMEM((2 * FEDGES,), jnp.int32),
            pltpu.VMEM((2 * FEDGES,), jnp.int32),
            pltpu.VMEM((CHUNK,), jnp.int32),
            pltpu.VMEM((CHUNK,), jnp.int32),
            pltpu.VMEM((CHUNK,), jnp.int32),
            pltpu.VMEM((CHUNK,), jnp.int32),
            pltpu.VMEM((CHUNK,), jnp.float32),
            pltpu.VMEM((ZCH,), jnp.float32),
            pltpu.SemaphoreType.DMA,
            pltpu.SemaphoreType.DMA,
            pltpu.SemaphoreType.DMA,
            pltpu.SemaphoreType.DMA,
        ],
    )


_SELU_ALPHA = 1.6732632423543772848170429916717
_SELU_SCALE = 1.0507009873554804934193349852946


GB = 4


def _xw_body(x_ref, W1r, o_ref):
    o_ref[...] = jnp.dot(x_ref[...], W1r[...],
                         preferred_element_type=jnp.float32)


def _xw(x, W1):
    return pl.pallas_call(
        _xw_body,
        grid_spec=pl.GridSpec(
            grid=(10,),
            in_specs=[pl.BlockSpec((1000, DIN), lambda g: (g, 0)),
                      pl.BlockSpec((DIN, HID), lambda g: (0, 0))],
            out_specs=pl.BlockSpec((1000, HID), lambda g: (g, 0)),
        ),
        out_shape=jax.ShapeDtypeStruct((10 * 1000, HID), jnp.float32),
    )(x, W1)


def _dense_body(adjL_ref, adjR_ref, xw_ref, b1r, War, bar, Wbr, bbr, Wrelr,
                brelr, Wrootr, Wl1r, bl1r, Wl2r, bl2r, out_ref):
    for b in range(GB):
        _dense_one(b, adjL_ref, adjR_ref, xw_ref, b1r, War, bar, Wbr, bbr,
                   Wrelr, brelr, Wrootr, Wl1r, bl1r, Wl2r, bl2r, out_ref)


def _dense_one(b, adjL_ref, adjR_ref, xw_ref, b1r, War, bar, Wbr, bbr, Wrelr,
               brelr, Wrootr, Wl1r, bl1r, Wl2r, bl2r, out_ref):
    f32 = jnp.float32
    L = adjL_ref[b * PAD:(b + 1) * PAD, :]
    R = adjR_ref[b * PAD:(b + 1) * PAD, :]
    xw = jnp.concatenate([xw_ref[b * NPGR:(b + 1) * NPGR, :],
                          jnp.zeros((PAD - NPGR, HID), f32)], axis=0)
    ones_n = jnp.ones((PAD, 1), f32)
    ones_k = jnp.ones((KC, 1), f32)
    rmask = (lax.broadcasted_iota(jnp.int32, (PAD, 1), 0) < NPGR).astype(f32)
    eye = (lax.broadcasted_iota(jnp.int32, (KC, KC), 0)
           == lax.broadcasted_iota(jnp.int32, (KC, KC), 1)).astype(f32)

    bf16 = jnp.bfloat16

    def dot(a, b, fast=False):
        if fast:
            a, b = a.astype(bf16), b.astype(bf16)
        return jnp.dot(a, b, preferred_element_type=f32)

    def dotT(a, b, fast=False):
        if fast:
            a, b = a.astype(bf16), b.astype(bf16)
        return lax.dot_general(a, b, (((0,), (0,)), ((), ())),
                               preferred_element_type=f32)

    deg = jnp.concatenate([dotT(L, ones_n), dotT(R, ones_n)], axis=0) + 1.0
    dinv = lax.rsqrt(deg)
    xwd = xw * dinv
    agg = jnp.concatenate([dotT(L, xwd), dotT(R, xwd)], axis=0)
    h = jax.nn.relu((agg + dinv * xw) * dinv + b1r[...]) * rmask

    spre = dot(dot(h, War[...]) + bar[...], Wbr[...]) + bbr[...]
    smax = jax.nn.softmax(spre, axis=-1)
    sm = smax * rmask
    outc = dotT(sm, h)
    outp = _SELU_SCALE * jnp.where(outc > 0, outc,
                                   _SELU_ALPHA * (jnp.exp(outc) - 1.0))
    t1 = dot(L, sm[0:128, :]) + dot(R, sm[128:PAD, :])
    out_adj = dotT(sm, t1)
    ones_h = jnp.ones((128, 1), f32)
    degr = dot(L, ones_h) + dot(R, ones_h)
    mtot = jnp.sum(degr) * 0.5
    ca = dotT(sm, degr)
    cb = dotT(degr, sm)
    normalizer = dot(ca, cb) * 0.5 / mtot
    spectral = -jnp.sum((out_adj - normalizer) * eye) * 0.5 / mtot
    ss = dotT(sm, sm)
    ss_norm = jnp.sqrt(jnp.sum(ss * ss))
    ortho = jnp.sqrt(jnp.sum((ss / ss_norm - eye * 0.25) ** 2))
    csz = dotT(sm, ones_n)
    cluster = jnp.sqrt(jnp.sum(csz * csz)) / NPGR * 4.0 - 1.0

    oa = out_adj * (1.0 - eye)
    r_col = dot(oa, ones_k)
    oaT = dotT(oa, eye)
    r_row = dotT(ones_k, oaT)
    dc_col = jnp.sqrt(r_col) + 1e-15
    dc_row = jnp.sqrt(r_row) + 1e-15
    oan = oa / dc_row / dc_col

    xc = dot(dot(oan, outp), Wrelr[...]) + brelr[...] + dot(outp, Wrootr[...])
    xgm = dotT(ones_k * (1.0 / KC), xc)
    logits = dot(jax.nn.relu(dot(xgm, Wl1r[...]) + bl1r[...]), Wl2r[...]) + bl2r[...]

    loss = spectral + ortho + cluster
    out_ref[b, 0:1, 0:5] = logits
    out_ref[b, 1:2, 0:1] = loss * jnp.ones((1, 1), f32)


def _dense(adj2, xw, b1, Wa, ba, Wb, bb, Wrel, brel, Wroot, Wl1, bl1, Wl2, bl2,
           interpret=False):
    full = lambda shape: pl.BlockSpec(shape, lambda g: (0,) * len(shape))
    nsteps = NG // GB
    grid_spec = pl.GridSpec(
        grid=(nsteps,),
        in_specs=[
            pl.BlockSpec((GB * PAD, 128), lambda g: (g, 0)),
            pl.BlockSpec((GB * PAD, 128), lambda g: (g + NG // GB, 0)),
            pl.BlockSpec((GB * NPGR, HID), lambda g: (g, 0)),
            full((1, HID)),
            full((HID, HID)), full((1, HID)),
            full((HID, KC)), full((1, KC)),
            full((HID, HID)), full((1, HID)),
            full((HID, HID)),
            full((HID, HID)), full((1, HID)),
            full((HID, 5)), full((1, 5)),
        ],
        out_specs=pl.BlockSpec((GB, 8, 128), lambda g: (g, 0, 0)),
    )
    return pl.pallas_call(
        _dense_body,
        grid_spec=grid_spec,
        out_shape=jax.ShapeDtypeStruct((NG, 8, 128), jnp.float32),
        interpret=interpret,
    )(adj2, adj2, xw, b1, Wa, ba, Wb, bb, Wrel, brel, Wroot, Wl1, bl1, Wl2, bl2)


def kernel(x, edge_index, batch, W1, b1, Wa, ba, Wb, bb, Wrel, brel, Wroot,
           Wl1, bl1, Wl2, bl2):
    xw = _xw(x, W1)
    adj2 = _adj_build_fn()(edge_index).reshape(2 * NG * PAD, 128)
    outb = _dense(adj2, xw, b1.reshape(1, -1), Wa, ba.reshape(1, -1),
                  Wb, bb.reshape(1, -1), Wrel, brel.reshape(1, -1), Wroot,
                  Wl1, bl1.reshape(1, -1), Wl2, bl2.reshape(1, -1))
    logits = outb[:, 0, :5]
    loss = jnp.mean(outb[:, 1, 0])
    return logits, loss


# speedup vs baseline: 1.0143x; 1.0143x over previous
import functools

import jax
import jax.numpy as jnp
from jax import lax
from jax.experimental import pallas as pl
from jax.experimental.pallas import tpu as pltpu
from jax.experimental.pallas import tpu_sc as plsc

NG = 40
NPGR = 250
PAD = 256
DIN = 128
HID = 128
KC = 16
EDGES = 320000

GHALF = NG // 2
SLOTS_HALF = GHALF * PAD * PAD
HALFW = GHALF * PAD * 128
DUMP = 8192
SC_WORDS = SLOTS_HALF + DUMP
NSUB = 16
CHUNK = 128
FEDGES = 4 * CHUNK
NFETCH = 39
EPT = NFETCH * FEDGES
ZCH = SC_WORDS // NSUB // 8
WB = SLOTS_HALF // NSUB
WCH = 8192
MAGIC = 134218


def _adj_body(e_hbm, out_hbm, acc_sh, eb0, eb1, idx0, idx1, idx2, idx3,
              ones_v, zer_v, sem_e0, sem_e1, sem_s, sem_z):
    c = lax.axis_index("c")
    s = lax.axis_index("s")
    idxs = (idx0, idx1, idx2, idx3)

    def _fill_z(i, _):
        zer_v[pl.ds(i * 16, 16)] = jnp.zeros((16,), jnp.float32)
        return ()
    lax.fori_loop(0, ZCH // 16, _fill_z, ())
    for i in range(CHUNK // 16):
        ones_v[pl.ds(i * 16, 16)] = jnp.ones((16,), jnp.float32)

    for k in range(8):
        pltpu.async_copy(zer_v, acc_sh.at[pl.ds(s * (8 * ZCH) + k * ZCH, ZCH)], sem_z)
    for k in range(8):
        pltpu.make_async_copy(zer_v, acc_sh.at[pl.ds(s * (8 * ZCH) + k * ZCH, ZCH)], sem_z).wait()
    plsc.subcore_barrier()

    glo = c * GHALF
    off_half = c * HALFW
    base = s * EPT

    def _start_fetch(f, eb, sem):
        pltpu.async_copy(e_hbm.at[0, pl.ds(base + f * FEDGES, FEDGES)],
                         eb.at[pl.ds(0, FEDGES)], sem)
        pltpu.async_copy(e_hbm.at[1, pl.ds(base + f * FEDGES, FEDGES)],
                         eb.at[pl.ds(FEDGES, FEDGES)], sem)

    def _wait_fetch(eb, sem):
        pltpu.make_async_copy(e_hbm.at[0, pl.ds(0, FEDGES)],
                              eb.at[pl.ds(0, FEDGES)], sem).wait()
        pltpu.make_async_copy(e_hbm.at[1, pl.ds(0, FEDGES)],
                              eb.at[pl.ds(FEDGES, FEDGES)], sem).wait()

    def _scatter_chunk(eb, koff, idx_c):
        for l in range(CHUNK // 16):
            sv = eb[pl.ds(koff + l * 16, 16)]
            dv = eb[pl.ds(FEDGES + koff + l * 16, 16)]
            g = lax.shift_right_logical(sv * MAGIC, 25)
            pd = dv - g * 250
            t = lax.shift_right_logical(pd, 7)
            flat = (t * HALFW + sv * 128 + g * 768 + (pd & 127)) - off_half
            ok = (g >= glo) & (g < glo + GHALF)
            idx_c[pl.ds(l * 16, 16)] = jnp.where(
                ok, flat, SLOTS_HALF + (sv & (DUMP - 1)))
        pltpu.async_copy(ones_v, acc_sh.at[idx_c], sem_s, add=True)

    def _process(eb, sem, f_next):
        _wait_fetch(eb, sem)
        for k in range(4):
            _scatter_chunk(eb, k * CHUNK, idxs[k])
        _start_fetch(f_next, eb, sem)
        for k in range(4):
            pltpu.make_async_copy(ones_v, acc_sh.at[idxs[k]], sem_s).wait()

    _start_fetch(0, eb0, sem_e0)
    _start_fetch(1, eb1, sem_e1)

    def _pair(p, _):
        f0 = 2 * p
        _process(eb0, sem_e0, jnp.minimum(f0 + 2, NFETCH - 1))
        _process(eb1, sem_e1, jnp.minimum(f0 + 3, NFETCH - 1))
        return ()
    lax.fori_loop(0, (NFETCH - 1) // 2, _pair, ())

    _process(eb0, sem_e0, NFETCH - 1)
    _wait_fetch(eb0, sem_e0)
    _wait_fetch(eb1, sem_e1)

    @pl.when(s < 4)
    def _tail():
        toff = NSUB * EPT + s * CHUNK
        pltpu.sync_copy(e_hbm.at[0, pl.ds(toff, CHUNK)], eb0.at[pl.ds(0, CHUNK)])
        pltpu.sync_copy(e_hbm.at[1, pl.ds(toff, CHUNK)],
                        eb0.at[pl.ds(FEDGES, CHUNK)])
        _scatter_chunk(eb0, 0, idx0)
        pltpu.make_async_copy(ones_v, acc_sh.at[idx0], sem_s).wait()

    plsc.subcore_barrier()

    WHH = WB // 2
    def _wb_ranges():
        for t in range(2):
            for k in range(WHH // WCH):
                loc = t * HALFW + s * WHH + k * WCH
                glob = t * (2 * HALFW) + c * HALFW + s * WHH + k * WCH
                yield loc, glob
    for loc, glob in _wb_ranges():
        pltpu.async_copy(acc_sh.at[pl.ds(loc, WCH)],
                         out_hbm.at[pl.ds(glob, WCH)], sem_z)
    for loc, glob in _wb_ranges():
        pltpu.make_async_copy(acc_sh.at[pl.ds(loc, WCH)],
                              out_hbm.at[pl.ds(glob, WCH)], sem_z).wait()


@functools.cache
def _adj_build_fn():
    return pl.kernel(
        _adj_body,
        out_type=jax.ShapeDtypeStruct((NG * PAD * PAD,), jnp.float32),
        mesh=plsc.VectorSubcoreMesh(core_axis_name="c", subcore_axis_name="s",
                                    num_cores=2, num_subcores=NSUB),
        scratch_types=[
            pltpu.VMEM_SHARED((SC_WORDS,), jnp.float32),
            pltpu.VMEM((2 * FEDGES,), jnp.int32),
            pltpu.VMEM((2 * FEDGES,), jnp.int32),
            pltpu.VMEM((CHUNK,), jnp.int32),
            pltpu.VMEM((CHUNK,), jnp.int32),
            pltpu.VMEM((CHUNK,), jnp.int32),
            pltpu.VMEM((CHUNK,), jnp.int32),
            pltpu.VMEM((CHUNK,), jnp.float32),
            pltpu.VMEM((ZCH,), jnp.float32),
            pltpu.SemaphoreType.DMA,
            pltpu.SemaphoreType.DMA,
            pltpu.SemaphoreType.DMA,
            pltpu.SemaphoreType.DMA,
        ],
    )


_SELU_ALPHA = 1.6732632423543772848170429916717
_SELU_SCALE = 1.0507009873554804934193349852946


GB = 4


def _xw_body(x_ref, W1r, o_ref):
    o_ref[...] = jnp.dot(x_ref[...], W1r[...],
                         preferred_element_type=jnp.float32)


def _xw(x, W1):
    return pl.pallas_call(
        _xw_body,
        grid_spec=pl.GridSpec(
            grid=(10,),
            in_specs=[pl.BlockSpec((1000, DIN), lambda g: (g, 0)),
                      pl.BlockSpec((DIN, HID), lambda g: (0, 0))],
            out_specs=pl.BlockSpec((1000, HID), lambda g: (g, 0)),
        ),
        out_shape=jax.ShapeDtypeStruct((10 * 1000, HID), jnp.float32),
    )(x, W1)


def _dense_body(adjL_ref, adjR_ref, xw_ref, b1r, War, bar, Wbr, bbr, Wrelr,
                brelr, Wrootr, Wl1r, bl1r, Wl2r, bl2r, out_ref):
    for b in range(GB):
        _dense_one(b, adjL_ref, adjR_ref, xw_ref, b1r, War, bar, Wbr, bbr,
                   Wrelr, brelr, Wrootr, Wl1r, bl1r, Wl2r, bl2r, out_ref)


def _dense_one(b, adjL_ref, adjR_ref, xw_ref, b1r, War, bar, Wbr, bbr, Wrelr,
               brelr, Wrootr, Wl1r, bl1r, Wl2r, bl2r, out_ref):
    f32 = jnp.float32
    L = adjL_ref[b * PAD:(b + 1) * PAD, :]
    R = adjR_ref[b * PAD:(b + 1) * PAD, :]
    xw = jnp.concatenate([xw_ref[b * NPGR:(b + 1) * NPGR, :],
                          jnp.zeros((PAD - NPGR, HID), f32)], axis=0)
    ones_n = jnp.ones((PAD, 1), f32)
    ones_k = jnp.ones((KC, 1), f32)
    rmask = (lax.broadcasted_iota(jnp.int32, (PAD, 1), 0) < NPGR).astype(f32)
    eye = (lax.broadcasted_iota(jnp.int32, (KC, KC), 0)
           == lax.broadcasted_iota(jnp.int32, (KC, KC), 1)).astype(f32)

    bf16 = jnp.bfloat16

    def dot(a, b, fast=False):
        if fast:
            a, b = a.astype(bf16), b.astype(bf16)
        return jnp.dot(a, b, preferred_element_type=f32)

    def dotT(a, b, fast=False):
        if fast:
            a, b = a.astype(bf16), b.astype(bf16)
        return lax.dot_general(a, b, (((0,), (0,)), ((), ())),
                               preferred_element_type=f32)

    deg = jnp.concatenate([dotT(L, ones_n), dotT(R, ones_n)], axis=0) + 1.0
    dinv = lax.rsqrt(deg)
    xwd = xw * dinv
    agg = jnp.concatenate([dotT(L, xwd), dotT(R, xwd)], axis=0)
    h = jax.nn.relu((agg + dinv * xw) * dinv + b1r[...]) * rmask

    spre = dot(dot(h, War[...]) + bar[...], Wbr[...]) + bbr[...]
    smax = jax.nn.softmax(spre, axis=-1)
    sm = smax * rmask
    outc = dotT(sm, h)
    outp = _SELU_SCALE * jnp.where(outc > 0, outc,
                                   _SELU_ALPHA * (jnp.exp(outc) - 1.0))
    t1 = dot(L, sm[0:128, :]) + dot(R, sm[128:PAD, :])
    out_adj = dotT(sm, t1)
    ones_h = jnp.ones((128, 1), f32)
    degr = dot(L, ones_h) + dot(R, ones_h)
    mtot = jnp.sum(degr) * 0.5
    ca = dotT(sm, degr)
    cb = dotT(degr, sm)
    normalizer = dot(ca, cb) * 0.5 / mtot
    spectral = -jnp.sum((out_adj - normalizer) * eye) * 0.5 / mtot
    ss = dotT(sm, sm)
    ss_norm = jnp.sqrt(jnp.sum(ss * ss))
    ortho = jnp.sqrt(jnp.sum((ss / ss_norm - eye * 0.25) ** 2))
    csz = dotT(sm, ones_n)
    cluster = jnp.sqrt(jnp.sum(csz * csz)) / NPGR * 4.0 - 1.0

    oa = out_adj * (1.0 - eye)
    r_col = dot(oa, ones_k)
    oaT = dotT(oa, eye)
    r_row = dotT(ones_k, oaT)
    dc_col = jnp.sqrt(r_col) + 1e-15
    dc_row = jnp.sqrt(r_row) + 1e-15
    oan = oa / dc_row / dc_col

    xc = dot(dot(oan, outp), Wrelr[...]) + brelr[...] + dot(outp, Wrootr[...])
    xgm = dotT(ones_k * (1.0 / KC), xc)
    logits = dot(jax.nn.relu(dot(xgm, Wl1r[...]) + bl1r[...]), Wl2r[...]) + bl2r[...]

    loss = spectral + ortho + cluster
    out_ref[b, 0:1, 0:5] = logits
    out_ref[b, 1:2, 0:1] = loss * jnp.ones((1, 1), f32)


def _dense(adj2, xw, b1, Wa, ba, Wb, bb, Wrel, brel, Wroot, Wl1, bl1, Wl2, bl2,
           interpret=False):
    full = lambda shape: pl.BlockSpec(shape, lambda g: (0,) * len(shape))
    nsteps = NG // GB
    grid_spec = pl.GridSpec(
        grid=(nsteps,),
        in_specs=[
            pl.BlockSpec((GB * PAD, 128), lambda g: (g, 0)),
            pl.BlockSpec((GB * PAD, 128), lambda g: (g + NG // GB, 0)),
            pl.BlockSpec((GB * NPGR, HID), lambda g: (g, 0)),
            full((1, HID)),
            full((HID, HID)), full((1, HID)),
            full((HID, KC)), full((1, KC)),
            full((HID, HID)), full((1, HID)),
            full((HID, HID)),
            full((HID, HID)), full((1, HID)),
            full((HID, 5)), full((1, 5)),
        ],
        out_specs=pl.BlockSpec((GB, 8, 128), lambda g: (g, 0, 0)),
    )
    return pl.pallas_call(
        _dense_body,
        grid_spec=grid_spec,
        out_shape=jax.ShapeDtypeStruct((NG, 8, 128), jnp.float32),
        interpret=interpret,
    )(adj2, adj2, xw, b1, Wa, ba, Wb, bb, Wrel, brel, Wroot, Wl1, bl1, Wl2, bl2)


def kernel(x, edge_index, batch, W1, b1, Wa, ba, Wb, bb, Wrel, brel, Wroot,
           Wl1, bl1, Wl2, bl2):
    xw = _xw(x, W1)
    adj2 = _adj_build_fn()(edge_index).reshape(2 * NG * PAD, 128)
    outb = _dense(adj2, xw, b1.reshape(1, -1), Wa, ba.reshape(1, -1),
                  Wb, bb.reshape(1, -1), Wrel, brel.reshape(1, -1), Wroot,
                  Wl1, bl1.reshape(1, -1), Wl2, bl2.reshape(1, -1))
    logits = outb[:, 0, :5]
    loss = jnp.mean(outb[:, 1, 0])
    return logits, loss
